# baseline (speedup 1.0000x reference)
import functools

import jax
import jax.numpy as jnp
from jax import lax
from jax.experimental import pallas as pl
from jax.experimental.pallas import tpu as pltpu
from jax.experimental.pallas import tpu_sc as plsc

B = 4096
C = 50
V = 1000000
A = 1000
D = 16

SR = 8
NC, NS = 2, 16
NW = NC * NS
SPW = B // NW
IPW = SPW * C
HB = C // 2
NBUF = 2


def _sc_bag_kernel(obsf_hbm, table3_hbm, out_hbm, idx_v, gbuf, acc_v, s0, s1):
    sems = (s0, s1)
    wid = lax.axis_index("s") * NC + lax.axis_index("c")
    pltpu.sync_copy(obsf_hbm.at[pl.ds(wid * IPW, IPW)], idx_v)

    def halfbag_ids(j2):
        base = j2 * HB
        v1 = idx_v[pl.ds(base, 16)]
        v2 = idx_v[pl.ds(base + 9, 16)]
        return [v1[t] if t < 16 else v2[t - 9] for t in range(HB)]

    def fire(j2, p):
        ids = halfbag_ids(j2)
        for t in range(HB):
            pltpu.async_copy(table3_hbm.at[ids[t] // SR], gbuf.at[p, t],
                             sems[p])

    def drain_acc(j2, p):
        for t in range(HB):
            pltpu.make_async_copy(table3_hbm.at[0], gbuf.at[p, t],
                                  sems[p]).wait()
        s = j2 // 2
        ids = halfbag_ids(j2)
        for t in range(HB):
            plsc.addupdate(acc_v.at[s], gbuf[p, t, ids[t] % SR, :])

    zero = jnp.zeros((D,), jnp.float32)
    for i in range(SPW):
        acc_v[i, :] = zero
    for p in range(NBUF):
        fire(p, p)

    def step(jj, _):
        for p in range(NBUF):
            j2 = jj * NBUF + p
            drain_acc(j2, p)

            @pl.when(j2 + NBUF < 2 * SPW)
            def _():
                fire(j2 + NBUF, p)
        return ()

    lax.fori_loop(0, 2 * SPW // NBUF, step, (), unroll=False)
    pltpu.sync_copy(acc_v, out_hbm.at[pl.ds(wid * SPW, SPW)])


@jax.jit
def _sc_bag(obs_flat, table3):
    mesh = plsc.VectorSubcoreMesh(core_axis_name="c", subcore_axis_name="s")
    return pl.kernel(
        _sc_bag_kernel,
        out_type=jax.ShapeDtypeStruct((B, D), jnp.float32),
        mesh=mesh,
        scratch_types=[
            pltpu.VMEM((IPW,), jnp.int32),
            pltpu.VMEM((NBUF, HB, SR, D), jnp.float32),
            pltpu.VMEM((SPW, D), jnp.float32),
            pltpu.SemaphoreType.DMA,
            pltpu.SemaphoreType.DMA,
        ],
        compiler_params=pltpu.CompilerParams(use_tc_tiling_on_sc=True),
    )(obs_flat, table3)


def _tc_combine_kernel(enc_ref, act_ref, atable_ref, w_ref, b_ref, out_ref):
    w1 = w_ref[0:1, 0:D]
    w2 = w_ref[0:1, D:2 * D]
    actproj = jnp.dot(atable_ref[...], w2.T,
                      preferred_element_type=jnp.float32)
    y_act = jnp.dot(act_ref[...], actproj,
                    preferred_element_type=jnp.float32)
    y_obs = jnp.dot(enc_ref[...], w1.T,
                    preferred_element_type=jnp.float32)
    out_ref[...] = y_obs * (1.0 / C) + y_act * (1.0 / A) + b_ref[0]


@jax.jit
def _tc_combine(enc, actions2d, act_table, W, b):
    g = 8
    bm = B // g
    return pl.pallas_call(
        _tc_combine_kernel,
        grid=(g,),
        in_specs=[
            pl.BlockSpec((bm, D), lambda i: (i, 0)),
            pl.BlockSpec((bm, A), lambda i: (i, 0)),
            pl.BlockSpec((A, D), lambda i: (0, 0)),
            pl.BlockSpec((1, 2 * D), lambda i: (0, 0)),
            pl.BlockSpec(memory_space=pltpu.SMEM),
        ],
        out_specs=pl.BlockSpec((bm, 1), lambda i: (i, 0)),
        out_shape=jax.ShapeDtypeStruct((B, 1), jnp.float32),
    )(enc, actions2d, act_table, W, b)


def kernel(observation, actions, obs_table, act_table, W, b):
    obs_flat = observation.astype(jnp.int32).reshape(B * C)
    table3 = obs_table.reshape(V // SR, SR, D)
    enc = _sc_bag(obs_flat, table3)
    return _tc_combine(enc, actions.reshape(B, A), act_table, W, b)


# speedup vs baseline: 1.0354x; 1.0354x over previous
import functools

import jax
import jax.numpy as jnp
from jax import lax
from jax.experimental import pallas as pl
from jax.experimental.pallas import tpu as pltpu
from jax.experimental.pallas import tpu_sc as plsc

B = 4096
C = 50
V = 1000000
A = 1000
D = 16

NC, NS = 2, 16
NW = NC * NS
SPW = B // NW
NBUF = 5


def _sc_bag_kernel(idx_hbm, table_hbm, out_hbm, idx_v, gbuf, acc_v,
                   s0, s1, s2, s3, s4):
    sems = (s0, s1, s2, s3, s4)
    wid = lax.axis_index("s") * NC + lax.axis_index("c")
    pltpu.sync_copy(idx_hbm.at[pl.ds(wid * C, C)], idx_v)

    def fire(j, p):
        pltpu.async_copy(table_hbm.at[idx_v.at[j]], gbuf.at[p], sems[p])

    def wait(j, p):
        pltpu.make_async_copy(table_hbm.at[idx_v.at[j]], gbuf.at[p],
                              sems[p]).wait()

    zero = jnp.zeros((D,), jnp.float32)
    for i in range(SPW):
        acc_v[i, :] = zero
    for p in range(NBUF):
        fire(p, p)

    def step(jj, _):
        for p in range(NBUF):
            j = jj * NBUF + p
            wait(j, p)
            for i in range(SPW):
                plsc.addupdate(acc_v.at[i], gbuf[p, i, :])

            @pl.when(j + NBUF < C)
            def _():
                fire(j + NBUF, p)
        return ()

    lax.fori_loop(0, C // NBUF, step, (), unroll=False)
    pltpu.sync_copy(acc_v, out_hbm.at[pl.ds(wid * SPW, SPW)])


@jax.jit
def _sc_bag(idx_perm, table):
    mesh = plsc.VectorSubcoreMesh(core_axis_name="c", subcore_axis_name="s")
    return pl.kernel(
        _sc_bag_kernel,
        out_type=jax.ShapeDtypeStruct((B, D), jnp.float32),
        mesh=mesh,
        scratch_types=[
            pltpu.VMEM((C, SPW), jnp.int32),
            pltpu.VMEM((NBUF, SPW, D), jnp.float32),
            pltpu.VMEM((SPW, D), jnp.float32),
            pltpu.SemaphoreType.DMA,
            pltpu.SemaphoreType.DMA,
            pltpu.SemaphoreType.DMA,
            pltpu.SemaphoreType.DMA,
            pltpu.SemaphoreType.DMA,
        ],
        compiler_params=pltpu.CompilerParams(use_tc_tiling_on_sc=False),
    )(idx_perm, table)


def _tc_combine_kernel(enc_ref, act_ref, atable_ref, w_ref, b_ref, out_ref):
    w1 = w_ref[0:1, 0:D]
    w2 = w_ref[0:1, D:2 * D]
    actproj = jnp.dot(atable_ref[...], w2.T,
                      preferred_element_type=jnp.float32)
    y_act = jnp.dot(act_ref[...], actproj,
                    preferred_element_type=jnp.float32)
    y_obs = jnp.dot(enc_ref[...], w1.T,
                    preferred_element_type=jnp.float32)
    out_ref[...] = y_obs * (1.0 / C) + y_act * (1.0 / A) + b_ref[0]


@jax.jit
def _tc_combine(enc, actions2d, act_table, W, b):
    g = 8
    bm = B // g
    return pl.pallas_call(
        _tc_combine_kernel,
        grid=(g,),
        in_specs=[
            pl.BlockSpec((bm, D), lambda i: (i, 0)),
            pl.BlockSpec((bm, A), lambda i: (i, 0)),
            pl.BlockSpec((A, D), lambda i: (0, 0)),
            pl.BlockSpec((1, 2 * D), lambda i: (0, 0)),
            pl.BlockSpec(memory_space=pltpu.SMEM),
        ],
        out_specs=pl.BlockSpec((bm, 1), lambda i: (i, 0)),
        out_shape=jax.ShapeDtypeStruct((B, 1), jnp.float32),
    )(enc, actions2d, act_table, W, b)


def kernel(observation, actions, obs_table, act_table, W, b):
    idx_perm = (observation.astype(jnp.int32)
                .reshape(NW, SPW, C).transpose(0, 2, 1).reshape(NW * C, SPW))
    enc = _sc_bag(idx_perm, obs_table)
    return _tc_combine(enc, actions.reshape(B, A), act_table, W, b)


# speedup vs baseline: 1.0787x; 1.0419x over previous
import functools

import jax
import jax.numpy as jnp
from jax import lax
from jax.experimental import pallas as pl
from jax.experimental.pallas import tpu as pltpu
from jax.experimental.pallas import tpu_sc as plsc

B = 4096
C = 50
V = 1000000
A = 1000
D = 16

NC, NS = 2, 16
NW = NC * NS
SAMPLES_PER_W = B // NW
CHUNK_SAMPLES = 2
CHUNK_ROWS = CHUNK_SAMPLES * C
NCHUNK = SAMPLES_PER_W // CHUNK_SAMPLES
NBUF = 4


def _tree_sum(vals):
    while len(vals) > 1:
        vals = [vals[i] + vals[i + 1] if i + 1 < len(vals) else vals[i]
                for i in range(0, len(vals), 2)]
    return vals[0]


def _sc_bag_kernel(obs2d_hbm, table_hbm, out_hbm, idx_v, rows_v, enc_v,
                   s0, s1, s2, s3):
    sems = (s0, s1, s2, s3)
    wid = lax.axis_index("s") * NC + lax.axis_index("c")
    pltpu.sync_copy(obs2d_hbm.at[pl.ds(wid * NCHUNK, NCHUNK)], idx_v)

    def fire(j, p):
        pltpu.async_copy(table_hbm.at[idx_v.at[j]], rows_v.at[p], sems[p])

    def wait(j, p):
        pltpu.make_async_copy(table_hbm.at[idx_v.at[j]], rows_v.at[p],
                              sems[p]).wait()

    for p in range(NBUF):
        fire(p, p)

    def step(jj, _):
        for p in range(NBUF):
            j = jj * NBUF + p
            wait(j, p)
            rows = rows_v.at[p]
            for s in range(CHUNK_SAMPLES):
                acc = _tree_sum([rows[s * C + c, :] for c in range(C)])
                enc_v[j * CHUNK_SAMPLES + s, :] = acc

            @pl.when(j + NBUF < NCHUNK)
            def _():
                fire(j + NBUF, p)
        return ()

    lax.fori_loop(0, NCHUNK // NBUF, step, (), unroll=False)
    pltpu.sync_copy(enc_v, out_hbm.at[pl.ds(wid * SAMPLES_PER_W,
                                            SAMPLES_PER_W)])


@jax.jit
def _sc_bag(obs2d, table):
    mesh = plsc.VectorSubcoreMesh(core_axis_name="c", subcore_axis_name="s")
    return pl.kernel(
        _sc_bag_kernel,
        out_type=jax.ShapeDtypeStruct((B, D), jnp.float32),
        mesh=mesh,
        scratch_types=[
            pltpu.VMEM((NCHUNK, CHUNK_ROWS), jnp.int32),
            pltpu.VMEM((NBUF, CHUNK_ROWS, D), jnp.float32),
            pltpu.VMEM((SAMPLES_PER_W, D), jnp.float32),
            pltpu.SemaphoreType.DMA,
            pltpu.SemaphoreType.DMA,
            pltpu.SemaphoreType.DMA,
            pltpu.SemaphoreType.DMA,
        ],
        compiler_params=pltpu.CompilerParams(use_tc_tiling_on_sc=False),
    )(obs2d, table)


def _tc_combine_kernel(enc_ref, act_ref, atable_ref, w_ref, b_ref, out_ref):
    w1 = w_ref[0:1, 0:D]
    w2 = w_ref[0:1, D:2 * D]
    actproj = jnp.dot(atable_ref[...], w2.T,
                      preferred_element_type=jnp.float32)
    y_act = jnp.dot(act_ref[...], actproj,
                    preferred_element_type=jnp.float32)
    y_obs = jnp.dot(enc_ref[...], w1.T,
                    preferred_element_type=jnp.float32)
    out_ref[...] = y_obs * (1.0 / C) + y_act * (1.0 / A) + b_ref[0]


@jax.jit
def _tc_combine(enc, actions2d, act_table, W, b):
    g = 8
    bm = B // g
    return pl.pallas_call(
        _tc_combine_kernel,
        grid=(g,),
        in_specs=[
            pl.BlockSpec((bm, D), lambda i: (i, 0)),
            pl.BlockSpec((bm, A), lambda i: (i, 0)),
            pl.BlockSpec((A, D), lambda i: (0, 0)),
            pl.BlockSpec((1, 2 * D), lambda i: (0, 0)),
            pl.BlockSpec(memory_space=pltpu.SMEM),
        ],
        out_specs=pl.BlockSpec((bm, 1), lambda i: (i, 0)),
        out_shape=jax.ShapeDtypeStruct((B, 1), jnp.float32),
    )(enc, actions2d, act_table, W, b)


def kernel(observation, actions, obs_table, act_table, W, b):
    obs2d = observation.astype(jnp.int32).reshape(B * C // CHUNK_ROWS,
                                                  CHUNK_ROWS)
    enc = _sc_bag(obs2d, obs_table)
    actions2d = actions.reshape(B, A)
    return _tc_combine(enc, actions2d, act_table, W, b)


# speedup vs baseline: 1.1029x; 1.0224x over previous
import functools

import jax
import jax.numpy as jnp
from jax import lax
from jax.experimental import pallas as pl
from jax.experimental.pallas import tpu as pltpu
from jax.experimental.pallas import tpu_sc as plsc

B = 4096
C = 50
V = 1000000
A = 1000
D = 16

NC, NS = 2, 16
NW = NC * NS
SAMPLES_PER_W = B // NW
CHUNK_SAMPLES = 2
CHUNK_ROWS = CHUNK_SAMPLES * C
NCHUNK = SAMPLES_PER_W // CHUNK_SAMPLES
NBUF = 4


def _tree_sum(vals):
    while len(vals) > 1:
        vals = [vals[i] + vals[i + 1] if i + 1 < len(vals) else vals[i]
                for i in range(0, len(vals), 2)]
    return vals[0]


def _sc_bag_kernel(obs2d_hbm, table_hbm, out_hbm, idx_v, rows_v, enc_v,
                   s0, s1, s2, s3):
    sems = (s0, s1, s2, s3)
    wid = lax.axis_index("s") * NC + lax.axis_index("c")
    pltpu.sync_copy(obs2d_hbm.at[pl.ds(wid * NCHUNK, NCHUNK)], idx_v)

    def fire(j, p):
        pltpu.async_copy(table_hbm.at[idx_v.at[j]], rows_v.at[p], sems[p])

    def wait(j, p):
        pltpu.make_async_copy(table_hbm.at[idx_v.at[j]], rows_v.at[p],
                              sems[p]).wait()

    for p in range(NBUF):
        fire(p, p)

    def step(jj, _):
        for p in range(NBUF):
            j = jj * NBUF + p
            wait(j, p)
            rows = rows_v.at[p]
            for s in range(CHUNK_SAMPLES):
                acc = _tree_sum([rows[s * C + c, :] for c in range(C)])
                enc_v[j * CHUNK_SAMPLES + s, :] = acc

            @pl.when(j + NBUF < NCHUNK)
            def _():
                fire(j + NBUF, p)
        return ()

    lax.fori_loop(0, NCHUNK // NBUF, step, (), unroll=False)
    pltpu.sync_copy(enc_v, out_hbm.at[pl.ds(wid * SAMPLES_PER_W,
                                            SAMPLES_PER_W)])


@jax.jit
def _sc_bag(obs2d, table):
    mesh = plsc.VectorSubcoreMesh(core_axis_name="c", subcore_axis_name="s")
    return pl.kernel(
        _sc_bag_kernel,
        out_type=jax.ShapeDtypeStruct((B, D), jnp.float32),
        mesh=mesh,
        scratch_types=[
            pltpu.VMEM((NCHUNK, CHUNK_ROWS), jnp.int32),
            pltpu.VMEM((NBUF, CHUNK_ROWS, D), jnp.float32),
            pltpu.VMEM((SAMPLES_PER_W, D), jnp.float32),
            pltpu.SemaphoreType.DMA,
            pltpu.SemaphoreType.DMA,
            pltpu.SemaphoreType.DMA,
            pltpu.SemaphoreType.DMA,
        ],
        compiler_params=pltpu.CompilerParams(use_tc_tiling_on_sc=False),
    )(obs2d, table)


def _tc_act_kernel(act_ref, atable_ref, w_ref, b_ref, yact_ref):
    w2 = w_ref[0:1, D:2 * D]
    actproj = jnp.dot(atable_ref[...], w2.T,
                      preferred_element_type=jnp.float32)
    y_act = jnp.dot(act_ref[...], actproj,
                    preferred_element_type=jnp.float32)
    yact_ref[...] = y_act * (1.0 / A) + b_ref[0]


@jax.jit
def _tc_act(actions2d, act_table, W, b):
    g = 8
    bm = B // g
    return pl.pallas_call(
        _tc_act_kernel,
        grid=(g,),
        in_specs=[
            pl.BlockSpec((bm, A), lambda i: (i, 0)),
            pl.BlockSpec((A, D), lambda i: (0, 0)),
            pl.BlockSpec((1, 2 * D), lambda i: (0, 0)),
            pl.BlockSpec(memory_space=pltpu.SMEM),
        ],
        out_specs=pl.BlockSpec((bm, 1), lambda i: (i, 0)),
        out_shape=jax.ShapeDtypeStruct((B, 1), jnp.float32),
    )(actions2d, act_table, W, b)


def _tc_combine_kernel(enc_ref, yact_ref, w_ref, out_ref):
    w1 = w_ref[0:1, 0:D]
    y_obs = jnp.dot(enc_ref[...], w1.T,
                    preferred_element_type=jnp.float32)
    out_ref[...] = y_obs * (1.0 / C) + yact_ref[...]


@jax.jit
def _tc_combine(enc, y_act, W):
    return pl.pallas_call(
        _tc_combine_kernel,
        in_specs=[
            pl.BlockSpec((B, D), lambda: (0, 0)),
            pl.BlockSpec((B, 1), lambda: (0, 0)),
            pl.BlockSpec((1, 2 * D), lambda: (0, 0)),
        ],
        out_specs=pl.BlockSpec((B, 1), lambda: (0, 0)),
        out_shape=jax.ShapeDtypeStruct((B, 1), jnp.float32),
    )(enc, y_act, W)


def kernel(observation, actions, obs_table, act_table, W, b):
    obs2d = observation.astype(jnp.int32).reshape(B * C // CHUNK_ROWS,
                                                  CHUNK_ROWS)
    y_act = _tc_act(actions.reshape(B, A), act_table, W, b)
    enc = _sc_bag(obs2d, obs_table)
    return _tc_combine(enc, y_act, W)
